# baseline (speedup 1.0000x reference)
import functools

import jax
import jax.numpy as jnp
from jax import lax
from jax.experimental import pallas as pl


def _conv_kernel(x_ref, w1_ref, b1_ref, w2_ref, feat_ref, cas_ref, mag_ref):
    h = pl.program_id(1)
    xb = x_ref[0]
    T = feat_ref.shape[1]
    acc = jnp.dot(xb[0:T], w1_ref[0], preferred_element_type=jnp.float32)
    acc = acc + jnp.dot(xb[1:T + 1], w1_ref[1], preferred_element_type=jnp.float32)
    acc = acc + jnp.dot(xb[2:T + 2], w1_ref[2], preferred_element_type=jnp.float32)
    feat = jnp.maximum(acc + b1_ref[...], 0.0)
    feat_ref[0] = feat
    cas_p = jnp.dot(feat, w2_ref[...], preferred_element_type=jnp.float32)
    mag_p = jnp.sum(feat * feat, axis=1, keepdims=True)

    @pl.when(h == 0)
    def _init():
        cas_ref[0] = cas_p
        mag_ref[0] = mag_p

    @pl.when(h > 0)
    def _accum():
        cas_ref[0] = cas_ref[0] + cas_p
        mag_ref[0] = mag_ref[0] + mag_p


def _select_kernel(feat_ref, cas_ref, mag_ref,
                   sa_ref, sb_ref, fa_ref, fb_ref, csm_ref,
                   *, T, C, KA, KB):
    featb = feat_ref[0]
    casb = cas_ref[0]
    mcol = mag_ref[0]
    row_i = lax.broadcasted_iota(jnp.int32, (T, T), 0)
    col_i = lax.broadcasted_iota(jnp.int32, (T, T), 1)
    tie = row_i < col_i
    eye = (row_i == col_i).astype(jnp.float32)

    vmat = jnp.broadcast_to(mcol, (T, T))
    mrow = jnp.sum(vmat * eye, axis=0, keepdims=True)
    eq_tie = (vmat == mrow) & tie
    a_act = (vmat > mrow) | eq_tie
    a_bkg = (vmat < mrow) | eq_tie
    rank_a = jnp.sum(a_act.astype(jnp.float32), axis=0, keepdims=True)
    rank_b = jnp.sum(a_bkg.astype(jnp.float32), axis=0, keepdims=True)

    p_a = lax.broadcasted_iota(jnp.int32, (KA, T), 0).astype(jnp.float32)
    oh_a = (rank_a == p_a).astype(jnp.float32)
    fa_ref[0] = jnp.dot(oh_a, featb, preferred_element_type=jnp.float32)

    p_b = lax.broadcasted_iota(jnp.int32, (KB, T), 0).astype(jnp.float32)
    oh_b = (rank_b == p_b).astype(jnp.float32)
    fb_ref[0] = jnp.dot(oh_b, featb, preferred_element_type=jnp.float32)

    selb = (rank_b < KB).astype(jnp.float32)
    sb = jnp.dot(selb, casb, preferred_element_type=jnp.float32) / KB

    c_iota = lax.broadcasted_iota(jnp.int32, (1, C), 1)

    def body(c, acc):
        onec = (c_iota == c).astype(jnp.float32)
        vc = jnp.sum(casb * onec, axis=1, keepdims=True)
        vm = jnp.broadcast_to(vc, (T, T))
        vr = jnp.sum(vm * eye, axis=0, keepdims=True)
        a = (vm > vr) | ((vm == vr) & tie)
        r = jnp.sum(a.astype(jnp.float32), axis=0, keepdims=True)
        sel = (r < KA).astype(jnp.float32)
        s = jnp.dot(sel, vc, preferred_element_type=jnp.float32)
        return acc + s * onec

    sa = lax.fori_loop(0, C, body, jnp.zeros((1, C), jnp.float32)) / KA

    def softmax_rows(v):
        m = jnp.max(v, axis=1, keepdims=True)
        e = jnp.exp(v - m)
        return e / jnp.sum(e, axis=1, keepdims=True)

    sa_ref[0] = softmax_rows(sa)
    sb_ref[0] = softmax_rows(sb)
    csm_ref[0] = softmax_rows(casb)


def kernel(x, W1, b1, W2):
    B, T, D = x.shape
    HID = W1.shape[0]
    C = W2.shape[0]
    KA = T // 8
    KB = T // 6
    HT = min(512, HID)
    NH = HID // HT

    xpad = jnp.pad(x, ((0, 0), (1, 1), (0, 0)))
    w1r = jnp.transpose(W1, (2, 1, 0))
    b1r = jnp.reshape(b1, (1, HID))
    w2r = jnp.transpose(W2[:, :, 0])

    feat, cas, mag2 = pl.pallas_call(
        _conv_kernel,
        grid=(B, NH),
        in_specs=[
            pl.BlockSpec((1, T + 2, D), lambda b, h: (b, 0, 0)),
            pl.BlockSpec((3, D, HT), lambda b, h: (0, 0, h)),
            pl.BlockSpec((1, HT), lambda b, h: (0, h)),
            pl.BlockSpec((HT, C), lambda b, h: (h, 0)),
        ],
        out_specs=[
            pl.BlockSpec((1, T, HT), lambda b, h: (b, 0, h)),
            pl.BlockSpec((1, T, C), lambda b, h: (b, 0, 0)),
            pl.BlockSpec((1, T, 1), lambda b, h: (b, 0, 0)),
        ],
        out_shape=[
            jax.ShapeDtypeStruct((B, T, HID), jnp.float32),
            jax.ShapeDtypeStruct((B, T, C), jnp.float32),
            jax.ShapeDtypeStruct((B, T, 1), jnp.float32),
        ],
    )(xpad, w1r, b1r, w2r)

    sel_body = functools.partial(_select_kernel, T=T, C=C, KA=KA, KB=KB)
    sa, sb, fa, fb, csm = pl.pallas_call(
        sel_body,
        grid=(B,),
        in_specs=[
            pl.BlockSpec((1, T, HID), lambda b: (b, 0, 0)),
            pl.BlockSpec((1, T, C), lambda b: (b, 0, 0)),
            pl.BlockSpec((1, T, 1), lambda b: (b, 0, 0)),
        ],
        out_specs=[
            pl.BlockSpec((1, 1, C), lambda b: (b, 0, 0)),
            pl.BlockSpec((1, 1, C), lambda b: (b, 0, 0)),
            pl.BlockSpec((1, KA, HID), lambda b: (b, 0, 0)),
            pl.BlockSpec((1, KB, HID), lambda b: (b, 0, 0)),
            pl.BlockSpec((1, T, C), lambda b: (b, 0, 0)),
        ],
        out_shape=[
            jax.ShapeDtypeStruct((B, 1, C), jnp.float32),
            jax.ShapeDtypeStruct((B, 1, C), jnp.float32),
            jax.ShapeDtypeStruct((B, KA, HID), jnp.float32),
            jax.ShapeDtypeStruct((B, KB, HID), jnp.float32),
            jax.ShapeDtypeStruct((B, T, C), jnp.float32),
        ],
    )(feat, cas, mag2)

    score_act = jnp.reshape(sa, (B, C))
    score_bkg = jnp.reshape(sb, (B, C))
    return (score_act, score_bkg, fa, fb, feat, csm)


# speedup vs baseline: 1.0371x; 1.0371x over previous
import functools

import jax
import jax.numpy as jnp
from jax import lax
from jax.experimental import pallas as pl


def _conv_kernel(x_ref, w1_ref, b1_ref, feat_ref):
    xb = x_ref[0]
    T = feat_ref.shape[1]
    acc = jnp.dot(xb[0:T], w1_ref[0], preferred_element_type=jnp.float32)
    acc = acc + jnp.dot(xb[1:T + 1], w1_ref[1], preferred_element_type=jnp.float32)
    acc = acc + jnp.dot(xb[2:T + 2], w1_ref[2], preferred_element_type=jnp.float32)
    feat_ref[0] = jnp.maximum(acc + b1_ref[...], 0.0)


def _select_kernel(feat_ref, w2_ref,
                   sa_ref, sb_ref, fa_ref, fb_ref, csm_ref,
                   *, T, C, KA, KB):
    featb = feat_ref[0]
    casb = jnp.dot(featb, w2_ref[...], preferred_element_type=jnp.float32)
    mcol = jnp.sum(featb * featb, axis=1, keepdims=True)
    row_i = lax.broadcasted_iota(jnp.int32, (T, T), 0)
    col_i = lax.broadcasted_iota(jnp.int32, (T, T), 1)
    tie = row_i < col_i
    eye = (row_i == col_i).astype(jnp.float32)

    vmat = jnp.broadcast_to(mcol, (T, T))
    mrow = jnp.sum(vmat * eye, axis=0, keepdims=True)
    eq_tie = (vmat == mrow) & tie
    a_act = (vmat > mrow) | eq_tie
    a_bkg = (vmat < mrow) | eq_tie
    rank_a = jnp.sum(a_act.astype(jnp.float32), axis=0, keepdims=True)
    rank_b = jnp.sum(a_bkg.astype(jnp.float32), axis=0, keepdims=True)

    p_a = lax.broadcasted_iota(jnp.int32, (KA, T), 0).astype(jnp.float32)
    oh_a = (rank_a == p_a).astype(jnp.float32)
    fa_ref[0] = jnp.dot(oh_a, featb, preferred_element_type=jnp.float32)

    p_b = lax.broadcasted_iota(jnp.int32, (KB, T), 0).astype(jnp.float32)
    oh_b = (rank_b == p_b).astype(jnp.float32)
    fb_ref[0] = jnp.dot(oh_b, featb, preferred_element_type=jnp.float32)

    selb = (rank_b < KB).astype(jnp.float32)
    sb = jnp.dot(selb, casb, preferred_element_type=jnp.float32) / KB

    c_iota = lax.broadcasted_iota(jnp.int32, (1, C), 1)

    def body(c, acc):
        onec = (c_iota == c).astype(jnp.float32)
        vc = jnp.sum(casb * onec, axis=1, keepdims=True)
        vm = jnp.broadcast_to(vc, (T, T))
        vr = jnp.sum(vm * eye, axis=0, keepdims=True)
        a = (vm > vr) | ((vm == vr) & tie)
        r = jnp.sum(a.astype(jnp.float32), axis=0, keepdims=True)
        sel = (r < KA).astype(jnp.float32)
        s = jnp.dot(sel, vc, preferred_element_type=jnp.float32)
        return acc + s * onec

    sa = lax.fori_loop(0, C, body, jnp.zeros((1, C), jnp.float32)) / KA

    def softmax_rows(v):
        m = jnp.max(v, axis=1, keepdims=True)
        e = jnp.exp(v - m)
        return e / jnp.sum(e, axis=1, keepdims=True)

    sa_ref[0] = softmax_rows(sa)
    sb_ref[0] = softmax_rows(sb)
    csm_ref[0] = softmax_rows(casb)


def kernel(x, W1, b1, W2):
    B, T, D = x.shape
    HID = W1.shape[0]
    C = W2.shape[0]
    KA = T // 8
    KB = T // 6
    HT = min(512, HID)
    NH = HID // HT

    xpad = jnp.pad(x, ((0, 0), (1, 1), (0, 0)))
    w1r = jnp.transpose(W1, (2, 1, 0))
    b1r = jnp.reshape(b1, (1, HID))
    w2r = jnp.transpose(W2[:, :, 0])

    feat = pl.pallas_call(
        _conv_kernel,
        grid=(NH, B),
        in_specs=[
            pl.BlockSpec((1, T + 2, D), lambda h, b: (b, 0, 0)),
            pl.BlockSpec((3, D, HT), lambda h, b: (0, 0, h)),
            pl.BlockSpec((1, HT), lambda h, b: (0, h)),
        ],
        out_specs=pl.BlockSpec((1, T, HT), lambda h, b: (b, 0, h)),
        out_shape=jax.ShapeDtypeStruct((B, T, HID), jnp.float32),
    )(xpad, w1r, b1r)

    sel_body = functools.partial(_select_kernel, T=T, C=C, KA=KA, KB=KB)
    sa, sb, fa, fb, csm = pl.pallas_call(
        sel_body,
        grid=(B,),
        in_specs=[
            pl.BlockSpec((1, T, HID), lambda b: (b, 0, 0)),
            pl.BlockSpec((HID, C), lambda b: (0, 0)),
        ],
        out_specs=[
            pl.BlockSpec((1, 1, C), lambda b: (b, 0, 0)),
            pl.BlockSpec((1, 1, C), lambda b: (b, 0, 0)),
            pl.BlockSpec((1, KA, HID), lambda b: (b, 0, 0)),
            pl.BlockSpec((1, KB, HID), lambda b: (b, 0, 0)),
            pl.BlockSpec((1, T, C), lambda b: (b, 0, 0)),
        ],
        out_shape=[
            jax.ShapeDtypeStruct((B, 1, C), jnp.float32),
            jax.ShapeDtypeStruct((B, 1, C), jnp.float32),
            jax.ShapeDtypeStruct((B, KA, HID), jnp.float32),
            jax.ShapeDtypeStruct((B, KB, HID), jnp.float32),
            jax.ShapeDtypeStruct((B, T, C), jnp.float32),
        ],
    )(feat, w2r)

    score_act = jnp.reshape(sa, (B, C))
    score_bkg = jnp.reshape(sb, (B, C))
    return (score_act, score_bkg, fa, fb, feat, csm)
